# baseline (speedup 1.0000x reference)
import functools

import jax
import jax.numpy as jnp
from jax import lax
from jax.experimental import pallas as pl
from jax.experimental.pallas import tpu as pltpu
from jax.experimental.pallas import tpu_sc as plsc

HIDDEN = 1024
NUM_EXPERTS = 8
TOPK = 2
FF = 2048
T = 2048
EPS = 1e-6
CAP = (T * TOPK) // NUM_EXPERTS
NSLOT = NUM_EXPERTS * CAP
XE_PAD = T + 16
LANES = 128


def _router_body(x_ref, gs_ref, wg_ref, gamma_ref, xe_ref, meta_ref):
    x = x_ref[...]
    ms = jnp.mean(x * x, axis=1, keepdims=True)
    r = lax.rsqrt(ms + EPS)
    xn = x * r
    xe_ref[pl.ds(0, T), :] = xn * gamma_ref[...]
    xe_ref[pl.ds(T, XE_PAD - T), :] = jnp.zeros((XE_PAD - T, HIDDEN), jnp.float32)

    xg = xn * (HIDDEN ** -0.5) * gs_ref[...]
    logits = jnp.dot(xg, wg_ref[...], preferred_element_type=jnp.float32)
    col = lax.broadcasted_iota(jnp.int32, (T, LANES), 1)
    valid_col = col < NUM_EXPERTS
    lg = jnp.where(valid_col, logits, -1e30)
    m = jnp.max(lg, axis=1, keepdims=True)
    e = jnp.where(valid_col, jnp.exp(lg - m), 0.0)
    p = e / jnp.sum(e, axis=1, keepdims=True)

    m0 = jnp.max(p, axis=1, keepdims=True)
    i0 = jnp.min(jnp.where(p >= m0, col, LANES), axis=1, keepdims=True)
    p1 = jnp.where(col == i0, -1.0, p)
    m1 = jnp.max(p1, axis=1, keepdims=True)
    i1 = jnp.min(jnp.where(p1 >= m1, col, LANES), axis=1, keepdims=True)
    wsum = jnp.maximum(m0 + m1, 1e-20)
    w0 = m0 / wsum
    w1 = m1 / wsum

    cnt = (col == i0).astype(jnp.float32) + (col == i1).astype(jnp.float32)
    c = cnt
    sh = 1
    while sh < T:
        z = jnp.zeros((sh, LANES), jnp.float32)
        c = c + jnp.concatenate([z, c[: T - sh, :]], axis=0)
        sh *= 2
    excl = c - cnt

    pos0 = jnp.sum(jnp.where(col == i0, excl, 0.0), axis=1, keepdims=True)
    pos1 = jnp.sum(jnp.where(col == i1, excl, 0.0), axis=1, keepdims=True)
    keep0 = pos0 < CAP
    keep1 = pos1 < CAP
    slot0 = i0.astype(jnp.float32) * CAP + jnp.minimum(pos0, CAP - 1)
    slot1 = i1.astype(jnp.float32) * CAP + jnp.minimum(pos1, CAP - 1)
    w0 = jnp.where(keep0, w0, 0.0)
    w1 = jnp.where(keep1, w1, 0.0)
    v0 = keep0.astype(jnp.float32)
    v1 = keep1.astype(jnp.float32)

    meta = jnp.where(col == 0, slot0,
           jnp.where(col == 1, slot1,
           jnp.where(col == 2, w0,
           jnp.where(col == 3, w1,
           jnp.where(col == 4, v0,
           jnp.where(col == 5, v1, 0.0))))))
    meta_ref[...] = meta


def _router(x, gate_scale, wg_pad, gamma):
    return pl.pallas_call(
        _router_body,
        out_shape=(
            jax.ShapeDtypeStruct((XE_PAD, HIDDEN), jnp.float32),
            jax.ShapeDtypeStruct((T, LANES), jnp.float32),
        ),
    )(x, gate_scale, wg_pad, gamma)


def _sc_mesh():
    return plsc.VectorSubcoreMesh(core_axis_name="c", subcore_axis_name="s")


@functools.lru_cache(maxsize=None)
def _slot_map_kernel_fn():
    return pl.kernel(
        _slot_map_body,
        out_type=jax.ShapeDtypeStruct((NSLOT,), jnp.int32),
        mesh=_sc_mesh(),
        scratch_types=[
            pltpu.VMEM((T,), jnp.int32),
            pltpu.VMEM((T,), jnp.int32),
            pltpu.VMEM((T,), jnp.int32),
            pltpu.VMEM((T,), jnp.int32),
            pltpu.VMEM((NSLOT,), jnp.int32),
        ],
        compiler_params=pltpu.CompilerParams(needs_layout_passes=False),
    )


def _slot_map_body(slot0_hbm, slot1_hbm, v0_hbm, v1_hbm, src_hbm,
                   s0_v, s1_v, v0_v, v1_v, src_v):
    cid = lax.axis_index("c")
    sid = lax.axis_index("s")
    is0 = jnp.logical_and(cid == 0, sid == 0)

    @pl.when(is0)
    def _():
        pltpu.sync_copy(slot0_hbm, s0_v)
        pltpu.sync_copy(slot1_hbm, s1_v)
        pltpu.sync_copy(v0_hbm, v0_v)
        pltpu.sync_copy(v1_hbm, v1_v)

        def init(i, _):
            src_v[pl.ds(i * 16, 16)] = jnp.full((16,), T, jnp.int32)
            return 0

        lax.fori_loop(0, NSLOT // 16, init, 0)

        def scat(i, _):
            toks = lax.iota(jnp.int32, 16) + i * 16
            s0 = s0_v[pl.ds(i * 16, 16)]
            m0 = v0_v[pl.ds(i * 16, 16)] > 0
            plsc.store_scatter(src_v, [s0], toks, mask=m0)
            s1 = s1_v[pl.ds(i * 16, 16)]
            m1 = v1_v[pl.ds(i * 16, 16)] > 0
            plsc.store_scatter(src_v, [s1], toks, mask=m1)
            return 0

        lax.fori_loop(0, T // 16, scat, 0)
        pltpu.sync_copy(src_v, src_hbm)


NW = 32
ROWS_W = NSLOT // NW
DCHUNK = 32


@functools.lru_cache(maxsize=None)
def _dispatch_kernel_fn():
    return pl.kernel(
        _dispatch_body,
        out_type=jax.ShapeDtypeStruct((NSLOT, HIDDEN), jnp.float32),
        mesh=_sc_mesh(),
        scratch_types=[
            pltpu.VMEM((ROWS_W,), jnp.int32),
            pltpu.VMEM((DCHUNK, HIDDEN), jnp.float32),
            pltpu.VMEM((DCHUNK, HIDDEN), jnp.float32),
            pltpu.SemaphoreType.DMA,
            pltpu.SemaphoreType.DMA,
        ],
        compiler_params=pltpu.CompilerParams(needs_layout_passes=False),
    )


def _dispatch_body(xe_hbm, src_hbm, out_hbm, idx_v, rows_a, rows_b, sem_a, sem_b):
    cid = lax.axis_index("c")
    sid = lax.axis_index("s")
    wid = sid * 2 + cid
    base = wid * ROWS_W
    pltpu.sync_copy(src_hbm.at[pl.ds(base, ROWS_W)], idx_v)
    nrounds = ROWS_W // (2 * DCHUNK)
    for rr in range(nrounds):
        off = rr * 2 * DCHUNK
        cp_a = pltpu.async_copy(
            xe_hbm.at[idx_v.at[pl.ds(off, DCHUNK)]], rows_a, sem_a)
        cp_b = pltpu.async_copy(
            xe_hbm.at[idx_v.at[pl.ds(off + DCHUNK, DCHUNK)]], rows_b, sem_b)
        cp_a.wait()
        pltpu.sync_copy(rows_a, out_hbm.at[pl.ds(base + off, DCHUNK)])
        cp_b.wait()
        pltpu.sync_copy(rows_b, out_hbm.at[pl.ds(base + off + DCHUNK, DCHUNK)])


FBLK = 256
NFB = FF // FBLK


def _ffn_body(d_ref, wg_ref, wu_ref, wd_ref, out_ref):
    j = pl.program_id(1)
    d = d_ref[0]
    g = jnp.dot(d, wg_ref[0], preferred_element_type=jnp.float32)
    u = jnp.dot(d, wu_ref[0], preferred_element_type=jnp.float32)
    h = jax.nn.gelu(g) * u
    o = jnp.dot(h, wd_ref[0], preferred_element_type=jnp.float32)

    @pl.when(j == 0)
    def _():
        out_ref[0] = o

    @pl.when(j > 0)
    def _():
        out_ref[0] = out_ref[0] + o


def _ffn(disp, w_gate, w_up, w_down):
    return pl.pallas_call(
        _ffn_body,
        grid=(NUM_EXPERTS, NFB),
        in_specs=[
            pl.BlockSpec((1, CAP, HIDDEN), lambda e, j: (e, 0, 0)),
            pl.BlockSpec((1, HIDDEN, FBLK), lambda e, j: (e, 0, j)),
            pl.BlockSpec((1, HIDDEN, FBLK), lambda e, j: (e, 0, j)),
            pl.BlockSpec((1, FBLK, HIDDEN), lambda e, j: (e, j, 0)),
        ],
        out_specs=pl.BlockSpec((1, CAP, HIDDEN), lambda e, j: (e, 0, 0)),
        out_shape=jax.ShapeDtypeStruct((NUM_EXPERTS, CAP, HIDDEN), jnp.float32),
        compiler_params=pltpu.CompilerParams(
            dimension_semantics=("arbitrary", "arbitrary"),
        ),
    )(disp, w_gate, w_up, w_down)


TOK_W = T // NW
CCHUNK = 32


@functools.lru_cache(maxsize=None)
def _combine_kernel_fn():
    return pl.kernel(
        _combine_body,
        out_type=jax.ShapeDtypeStruct((T, HIDDEN), jnp.float32),
        mesh=_sc_mesh(),
        scratch_types=[
            pltpu.VMEM((TOK_W,), jnp.int32),
            pltpu.VMEM((TOK_W,), jnp.int32),
            pltpu.VMEM((TOK_W * 16,), jnp.float32),
            pltpu.VMEM((TOK_W * 16,), jnp.float32),
            pltpu.VMEM((CCHUNK, HIDDEN), jnp.float32),
            pltpu.VMEM((CCHUNK, HIDDEN), jnp.float32),
            pltpu.VMEM((CCHUNK, HIDDEN), jnp.float32),
            pltpu.SemaphoreType.DMA,
            pltpu.SemaphoreType.DMA,
        ],
        compiler_params=pltpu.CompilerParams(needs_layout_passes=False),
    )


def _combine_body(eo_hbm, slot0_hbm, slot1_hbm, w0_hbm, w1_hbm, y_hbm,
                  i0_v, i1_v, w0_v, w1_v, r0_v, r1_v, y_v, sem0, sem1):
    cid = lax.axis_index("c")
    sid = lax.axis_index("s")
    wid = sid * 2 + cid
    base = wid * TOK_W
    pltpu.sync_copy(slot0_hbm.at[pl.ds(base, TOK_W)], i0_v)
    pltpu.sync_copy(slot1_hbm.at[pl.ds(base, TOK_W)], i1_v)
    pltpu.sync_copy(w0_hbm.at[pl.ds(base * 16, TOK_W * 16)], w0_v)
    pltpu.sync_copy(w1_hbm.at[pl.ds(base * 16, TOK_W * 16)], w1_v)

    for cchunk in range(TOK_W // CCHUNK):
        c0 = cchunk * CCHUNK
        cp0 = pltpu.async_copy(eo_hbm.at[i0_v.at[pl.ds(c0, CCHUNK)]], r0_v, sem0)
        cp1 = pltpu.async_copy(eo_hbm.at[i1_v.at[pl.ds(c0, CCHUNK)]], r1_v, sem1)
        cp0.wait()
        cp1.wait()

        def tok(i, _):
            w0 = w0_v[pl.ds((c0 + i) * 16, 16)]
            w1 = w1_v[pl.ds((c0 + i) * 16, 16)]
            for jj in range(HIDDEN // 16):
                a = r0_v[i, pl.ds(jj * 16, 16)]
                b = r1_v[i, pl.ds(jj * 16, 16)]
                y_v[i, pl.ds(jj * 16, 16)] = w0 * a + w1 * b
            return 0

        lax.fori_loop(0, CCHUNK, tok, 0)
        pltpu.sync_copy(y_v, y_hbm.at[pl.ds(base + c0, CCHUNK)])


def kernel(x, gate_scale, Wg, pre_norm_gamma, W_gate, W_up, W_down):
    wg_pad = jnp.zeros((HIDDEN, LANES), jnp.float32).at[:, :NUM_EXPERTS].set(Wg)
    xe, meta = _router(x, gate_scale.reshape(1, HIDDEN), wg_pad,
                       pre_norm_gamma.reshape(1, HIDDEN))
    slot0 = meta[:, 0].astype(jnp.int32)
    slot1 = meta[:, 1].astype(jnp.int32)
    w0 = jnp.broadcast_to(meta[:, 2:3], (T, 16)).reshape(-1)
    w1 = jnp.broadcast_to(meta[:, 3:4], (T, 16)).reshape(-1)
    v0 = meta[:, 4].astype(jnp.int32)
    v1 = meta[:, 5].astype(jnp.int32)

    src = _slot_map_kernel_fn()(slot0, slot1, v0, v1)
    disp = _dispatch_kernel_fn()(xe, src)
    eo = _ffn(disp.reshape(NUM_EXPERTS, CAP, HIDDEN), W_gate, W_up, W_down)
    y = _combine_kernel_fn()(eo.reshape(NSLOT, HIDDEN), slot0, slot1, w0, w1)
    return y


# speedup vs baseline: 1.1887x; 1.1887x over previous
import functools

import jax
import jax.numpy as jnp
from jax import lax
from jax.experimental import pallas as pl
from jax.experimental.pallas import tpu as pltpu
from jax.experimental.pallas import tpu_sc as plsc

HIDDEN = 1024
NUM_EXPERTS = 8
TOPK = 2
FF = 2048
T = 2048
EPS = 1e-6
CAP = (T * TOPK) // NUM_EXPERTS
NSLOT = NUM_EXPERTS * CAP
XE_PAD = T + 16
LANES = 128


def _router_body(x_ref, gs_ref, wg_ref, gamma_ref, xe_ref, meta_ref):
    x = x_ref[...]
    ms = jnp.mean(x * x, axis=1, keepdims=True)
    r = lax.rsqrt(ms + EPS)
    xn = x * r
    xe_ref[pl.ds(0, T), :] = xn * gamma_ref[...]
    xe_ref[pl.ds(T, XE_PAD - T), :] = jnp.zeros((XE_PAD - T, HIDDEN), jnp.float32)

    xg = xn * (HIDDEN ** -0.5) * gs_ref[...]
    logits = jnp.dot(xg, wg_ref[...], preferred_element_type=jnp.float32)
    col = lax.broadcasted_iota(jnp.int32, (T, LANES), 1)
    valid_col = col < NUM_EXPERTS
    lg = jnp.where(valid_col, logits, -1e30)
    m = jnp.max(lg, axis=1, keepdims=True)
    e = jnp.where(valid_col, jnp.exp(lg - m), 0.0)
    p = e / jnp.sum(e, axis=1, keepdims=True)

    m0 = jnp.max(p, axis=1, keepdims=True)
    i0 = jnp.min(jnp.where(p >= m0, col, LANES), axis=1, keepdims=True)
    p1 = jnp.where(col == i0, -1.0, p)
    m1 = jnp.max(p1, axis=1, keepdims=True)
    i1 = jnp.min(jnp.where(p1 >= m1, col, LANES), axis=1, keepdims=True)
    wsum = jnp.maximum(m0 + m1, 1e-20)
    w0 = m0 / wsum
    w1 = m1 / wsum

    cnt = (col == i0).astype(jnp.float32) + (col == i1).astype(jnp.float32)
    c = cnt
    sh = 1
    while sh < T:
        z = jnp.zeros((sh, LANES), jnp.float32)
        c = c + jnp.concatenate([z, c[: T - sh, :]], axis=0)
        sh *= 2
    excl = c - cnt

    pos0 = jnp.sum(jnp.where(col == i0, excl, 0.0), axis=1, keepdims=True)
    pos1 = jnp.sum(jnp.where(col == i1, excl, 0.0), axis=1, keepdims=True)
    keep0 = pos0 < CAP
    keep1 = pos1 < CAP
    slot0 = i0.astype(jnp.float32) * CAP + jnp.minimum(pos0, CAP - 1)
    slot1 = i1.astype(jnp.float32) * CAP + jnp.minimum(pos1, CAP - 1)
    w0 = jnp.where(keep0, w0, 0.0)
    w1 = jnp.where(keep1, w1, 0.0)
    v0 = keep0.astype(jnp.float32)
    v1 = keep1.astype(jnp.float32)

    meta = jnp.where(col == 0, slot0,
           jnp.where(col == 1, slot1,
           jnp.where(col == 2, w0,
           jnp.where(col == 3, w1,
           jnp.where(col == 4, v0,
           jnp.where(col == 5, v1, 0.0))))))
    meta_ref[...] = meta


def _router(x, gate_scale, wg_pad, gamma):
    return pl.pallas_call(
        _router_body,
        out_shape=(
            jax.ShapeDtypeStruct((XE_PAD, HIDDEN), jnp.float32),
            jax.ShapeDtypeStruct((T, LANES), jnp.float32),
        ),
    )(x, gate_scale, wg_pad, gamma)


def _sc_mesh():
    return plsc.VectorSubcoreMesh(core_axis_name="c", subcore_axis_name="s")


@functools.lru_cache(maxsize=None)
def _slot_map_kernel_fn():
    return pl.kernel(
        _slot_map_body,
        out_type=jax.ShapeDtypeStruct((NSLOT,), jnp.int32),
        mesh=_sc_mesh(),
        scratch_types=[
            pltpu.VMEM((T,), jnp.int32),
            pltpu.VMEM((T,), jnp.int32),
            pltpu.VMEM((T,), jnp.int32),
            pltpu.VMEM((T,), jnp.int32),
            pltpu.VMEM((NSLOT,), jnp.int32),
        ],
        compiler_params=pltpu.CompilerParams(needs_layout_passes=False),
    )


def _slot_map_body(slot0_hbm, slot1_hbm, v0_hbm, v1_hbm, src_hbm,
                   s0_v, s1_v, v0_v, v1_v, src_v):
    cid = lax.axis_index("c")
    sid = lax.axis_index("s")
    is0 = jnp.logical_and(cid == 0, sid == 0)

    @pl.when(is0)
    def _():
        pltpu.sync_copy(slot0_hbm, s0_v)
        pltpu.sync_copy(slot1_hbm, s1_v)
        pltpu.sync_copy(v0_hbm, v0_v)
        pltpu.sync_copy(v1_hbm, v1_v)

        def init(i, _):
            src_v[pl.ds(i * 16, 16)] = jnp.full((16,), T, jnp.int32)
            return 0

        lax.fori_loop(0, NSLOT // 16, init, 0)

        def scat(i, _):
            toks = lax.iota(jnp.int32, 16) + i * 16
            s0 = s0_v[pl.ds(i * 16, 16)]
            m0 = v0_v[pl.ds(i * 16, 16)] > 0
            plsc.store_scatter(src_v, [s0], toks, mask=m0)
            s1 = s1_v[pl.ds(i * 16, 16)]
            m1 = v1_v[pl.ds(i * 16, 16)] > 0
            plsc.store_scatter(src_v, [s1], toks, mask=m1)
            return 0

        lax.fori_loop(0, T // 16, scat, 0)
        pltpu.sync_copy(src_v, src_hbm)


NW = 32
ROWS_W = NSLOT // NW
DCHUNK = 32


@functools.lru_cache(maxsize=None)
def _dispatch_kernel_fn():
    return pl.kernel(
        _dispatch_body,
        out_type=jax.ShapeDtypeStruct((NSLOT, HIDDEN), jnp.float32),
        mesh=_sc_mesh(),
        scratch_types=[
            pltpu.VMEM((ROWS_W,), jnp.int32),
            pltpu.VMEM((DCHUNK, HIDDEN), jnp.float32),
            pltpu.VMEM((DCHUNK, HIDDEN), jnp.float32),
            pltpu.SemaphoreType.DMA,
            pltpu.SemaphoreType.DMA,
        ],
        compiler_params=pltpu.CompilerParams(needs_layout_passes=False),
    )


def _dispatch_body(xe_hbm, src_hbm, out_hbm, idx_v, rows_a, rows_b, sem_a, sem_b):
    cid = lax.axis_index("c")
    sid = lax.axis_index("s")
    wid = sid * 2 + cid
    base = wid * ROWS_W
    pltpu.sync_copy(src_hbm.at[pl.ds(base, ROWS_W)], idx_v)
    nrounds = ROWS_W // (2 * DCHUNK)
    for rr in range(nrounds):
        off = rr * 2 * DCHUNK
        cp_a = pltpu.async_copy(
            xe_hbm.at[idx_v.at[pl.ds(off, DCHUNK)]], rows_a, sem_a)
        cp_b = pltpu.async_copy(
            xe_hbm.at[idx_v.at[pl.ds(off + DCHUNK, DCHUNK)]], rows_b, sem_b)
        cp_a.wait()
        pltpu.sync_copy(rows_a, out_hbm.at[pl.ds(base + off, DCHUNK)])
        cp_b.wait()
        pltpu.sync_copy(rows_b, out_hbm.at[pl.ds(base + off + DCHUNK, DCHUNK)])


FBLK = 1024
NFB = FF // FBLK


def _ffn_body(d_ref, wg_ref, wu_ref, wd_ref, out_ref):
    j = pl.program_id(1)
    d = d_ref[0]
    g = jnp.dot(d, wg_ref[0], preferred_element_type=jnp.float32)
    u = jnp.dot(d, wu_ref[0], preferred_element_type=jnp.float32)
    h = jax.nn.gelu(g) * u
    o = jnp.dot(h, wd_ref[0], preferred_element_type=jnp.float32)

    @pl.when(j == 0)
    def _():
        out_ref[0] = o

    @pl.when(j > 0)
    def _():
        out_ref[0] = out_ref[0] + o


def _ffn(disp, w_gate, w_up, w_down):
    return pl.pallas_call(
        _ffn_body,
        grid=(NUM_EXPERTS, NFB),
        in_specs=[
            pl.BlockSpec((1, CAP, HIDDEN), lambda e, j: (e, 0, 0)),
            pl.BlockSpec((1, HIDDEN, FBLK), lambda e, j: (e, 0, j)),
            pl.BlockSpec((1, HIDDEN, FBLK), lambda e, j: (e, 0, j)),
            pl.BlockSpec((1, FBLK, HIDDEN), lambda e, j: (e, j, 0)),
        ],
        out_specs=pl.BlockSpec((1, CAP, HIDDEN), lambda e, j: (e, 0, 0)),
        out_shape=jax.ShapeDtypeStruct((NUM_EXPERTS, CAP, HIDDEN), jnp.float32),
        compiler_params=pltpu.CompilerParams(
            dimension_semantics=("arbitrary", "arbitrary"),
        ),
    )(disp, w_gate, w_up, w_down)


TOK_W = T // NW
CCHUNK = 32


@functools.lru_cache(maxsize=None)
def _combine_kernel_fn():
    return pl.kernel(
        _combine_body,
        out_type=jax.ShapeDtypeStruct((T, HIDDEN), jnp.float32),
        mesh=_sc_mesh(),
        scratch_types=[
            pltpu.VMEM((TOK_W,), jnp.int32),
            pltpu.VMEM((TOK_W,), jnp.int32),
            pltpu.VMEM((TOK_W * 16,), jnp.float32),
            pltpu.VMEM((TOK_W * 16,), jnp.float32),
            pltpu.VMEM((CCHUNK, HIDDEN), jnp.float32),
            pltpu.VMEM((CCHUNK, HIDDEN), jnp.float32),
            pltpu.VMEM((CCHUNK, HIDDEN), jnp.float32),
            pltpu.SemaphoreType.DMA,
            pltpu.SemaphoreType.DMA,
        ],
        compiler_params=pltpu.CompilerParams(needs_layout_passes=False),
    )


def _combine_body(eo_hbm, slot0_hbm, slot1_hbm, w0_hbm, w1_hbm, y_hbm,
                  i0_v, i1_v, w0_v, w1_v, r0_v, r1_v, y_v, sem0, sem1):
    cid = lax.axis_index("c")
    sid = lax.axis_index("s")
    wid = sid * 2 + cid
    base = wid * TOK_W
    pltpu.sync_copy(slot0_hbm.at[pl.ds(base, TOK_W)], i0_v)
    pltpu.sync_copy(slot1_hbm.at[pl.ds(base, TOK_W)], i1_v)
    pltpu.sync_copy(w0_hbm.at[pl.ds(base * 16, TOK_W * 16)], w0_v)
    pltpu.sync_copy(w1_hbm.at[pl.ds(base * 16, TOK_W * 16)], w1_v)

    for cchunk in range(TOK_W // CCHUNK):
        c0 = cchunk * CCHUNK
        cp0 = pltpu.async_copy(eo_hbm.at[i0_v.at[pl.ds(c0, CCHUNK)]], r0_v, sem0)
        cp1 = pltpu.async_copy(eo_hbm.at[i1_v.at[pl.ds(c0, CCHUNK)]], r1_v, sem1)
        cp0.wait()
        cp1.wait()

        def tok(i, _):
            w0 = w0_v[pl.ds((c0 + i) * 16, 16)]
            w1 = w1_v[pl.ds((c0 + i) * 16, 16)]
            for jj in range(HIDDEN // 16):
                a = r0_v[i, pl.ds(jj * 16, 16)]
                b = r1_v[i, pl.ds(jj * 16, 16)]
                y_v[i, pl.ds(jj * 16, 16)] = w0 * a + w1 * b
            return 0

        lax.fori_loop(0, CCHUNK, tok, 0)
        pltpu.sync_copy(y_v, y_hbm.at[pl.ds(base + c0, CCHUNK)])


def kernel(x, gate_scale, Wg, pre_norm_gamma, W_gate, W_up, W_down):
    wg_pad = jnp.zeros((HIDDEN, LANES), jnp.float32).at[:, :NUM_EXPERTS].set(Wg)
    xe, meta = _router(x, gate_scale.reshape(1, HIDDEN), wg_pad,
                       pre_norm_gamma.reshape(1, HIDDEN))
    slot0 = meta[:, 0].astype(jnp.int32)
    slot1 = meta[:, 1].astype(jnp.int32)
    w0 = jnp.broadcast_to(meta[:, 2:3], (T, 16)).reshape(-1)
    w1 = jnp.broadcast_to(meta[:, 3:4], (T, 16)).reshape(-1)
    v0 = meta[:, 4].astype(jnp.int32)
    v1 = meta[:, 5].astype(jnp.int32)

    src = _slot_map_kernel_fn()(slot0, slot1, v0, v1)
    disp = _dispatch_kernel_fn()(xe, src)
    eo = _ffn(disp.reshape(NUM_EXPERTS, CAP, HIDDEN), W_gate, W_up, W_down)
    y = _combine_kernel_fn()(eo.reshape(NSLOT, HIDDEN), slot0, slot1, w0, w1)
    return y


# speedup vs baseline: 1.1888x; 1.0001x over previous
import functools

import jax
import jax.numpy as jnp
from jax import lax
from jax.experimental import pallas as pl
from jax.experimental.pallas import tpu as pltpu
from jax.experimental.pallas import tpu_sc as plsc

HIDDEN = 1024
NUM_EXPERTS = 8
TOPK = 2
FF = 2048
T = 2048
EPS = 1e-6
CAP = (T * TOPK) // NUM_EXPERTS
NSLOT = NUM_EXPERTS * CAP
XE_PAD = T + 16
LANES = 128


def _router_body(x_ref, gs_ref, wg_ref, gamma_ref, xe_ref, meta_ref):
    x = x_ref[...]
    ms = jnp.mean(x * x, axis=1, keepdims=True)
    r = lax.rsqrt(ms + EPS)
    xn = x * r
    xe_ref[pl.ds(0, T), :] = xn * gamma_ref[...]
    xe_ref[pl.ds(T, XE_PAD - T), :] = jnp.zeros((XE_PAD - T, HIDDEN), jnp.float32)

    xg = xn * (HIDDEN ** -0.5) * gs_ref[...]
    logits = jnp.dot(xg, wg_ref[...], preferred_element_type=jnp.float32)
    col = lax.broadcasted_iota(jnp.int32, (T, LANES), 1)
    valid_col = col < NUM_EXPERTS
    lg = jnp.where(valid_col, logits, -1e30)
    m = jnp.max(lg, axis=1, keepdims=True)
    e = jnp.where(valid_col, jnp.exp(lg - m), 0.0)
    p = e / jnp.sum(e, axis=1, keepdims=True)

    m0 = jnp.max(p, axis=1, keepdims=True)
    i0 = jnp.min(jnp.where(p >= m0, col, LANES), axis=1, keepdims=True)
    p1 = jnp.where(col == i0, -1.0, p)
    m1 = jnp.max(p1, axis=1, keepdims=True)
    i1 = jnp.min(jnp.where(p1 >= m1, col, LANES), axis=1, keepdims=True)
    wsum = jnp.maximum(m0 + m1, 1e-20)
    w0 = m0 / wsum
    w1 = m1 / wsum

    cnt = (col == i0).astype(jnp.float32) + (col == i1).astype(jnp.float32)
    c = cnt
    sh = 1
    while sh < T:
        z = jnp.zeros((sh, LANES), jnp.float32)
        c = c + jnp.concatenate([z, c[: T - sh, :]], axis=0)
        sh *= 2
    excl = c - cnt

    pos0 = jnp.sum(jnp.where(col == i0, excl, 0.0), axis=1, keepdims=True)
    pos1 = jnp.sum(jnp.where(col == i1, excl, 0.0), axis=1, keepdims=True)
    keep0 = pos0 < CAP
    keep1 = pos1 < CAP
    slot0 = i0.astype(jnp.float32) * CAP + jnp.minimum(pos0, CAP - 1)
    slot1 = i1.astype(jnp.float32) * CAP + jnp.minimum(pos1, CAP - 1)
    w0 = jnp.where(keep0, w0, 0.0)
    w1 = jnp.where(keep1, w1, 0.0)
    v0 = keep0.astype(jnp.float32)
    v1 = keep1.astype(jnp.float32)

    meta = jnp.where(col == 0, slot0,
           jnp.where(col == 1, slot1,
           jnp.where(col == 2, w0,
           jnp.where(col == 3, w1,
           jnp.where(col == 4, v0,
           jnp.where(col == 5, v1, 0.0))))))
    meta_ref[...] = meta


def _router(x, gate_scale, wg_pad, gamma):
    return pl.pallas_call(
        _router_body,
        out_shape=(
            jax.ShapeDtypeStruct((XE_PAD, HIDDEN), jnp.float32),
            jax.ShapeDtypeStruct((T, LANES), jnp.float32),
        ),
    )(x, gate_scale, wg_pad, gamma)


def _sc_mesh():
    return plsc.VectorSubcoreMesh(core_axis_name="c", subcore_axis_name="s")


@functools.lru_cache(maxsize=None)
def _slot_map_kernel_fn():
    return pl.kernel(
        _slot_map_body,
        out_type=jax.ShapeDtypeStruct((NSLOT,), jnp.int32),
        mesh=_sc_mesh(),
        scratch_types=[
            pltpu.VMEM((T,), jnp.int32),
            pltpu.VMEM((T,), jnp.int32),
            pltpu.VMEM((T,), jnp.int32),
            pltpu.VMEM((T,), jnp.int32),
            pltpu.VMEM((NSLOT,), jnp.int32),
        ],
        compiler_params=pltpu.CompilerParams(needs_layout_passes=False),
    )


def _slot_map_body(slot0_hbm, slot1_hbm, v0_hbm, v1_hbm, src_hbm,
                   s0_v, s1_v, v0_v, v1_v, src_v):
    cid = lax.axis_index("c")
    sid = lax.axis_index("s")
    is0 = jnp.logical_and(cid == 0, sid == 0)

    @pl.when(is0)
    def _():
        pltpu.sync_copy(slot0_hbm, s0_v)
        pltpu.sync_copy(slot1_hbm, s1_v)
        pltpu.sync_copy(v0_hbm, v0_v)
        pltpu.sync_copy(v1_hbm, v1_v)

        def init(i, _):
            src_v[pl.ds(i * 16, 16)] = jnp.full((16,), T, jnp.int32)
            return 0

        lax.fori_loop(0, NSLOT // 16, init, 0)

        def scat(i, _):
            toks = lax.iota(jnp.int32, 16) + i * 16
            s0 = s0_v[pl.ds(i * 16, 16)]
            m0 = v0_v[pl.ds(i * 16, 16)] > 0
            plsc.store_scatter(src_v, [s0], toks, mask=m0)
            s1 = s1_v[pl.ds(i * 16, 16)]
            m1 = v1_v[pl.ds(i * 16, 16)] > 0
            plsc.store_scatter(src_v, [s1], toks, mask=m1)
            return 0

        lax.fori_loop(0, T // 16, scat, 0)
        pltpu.sync_copy(src_v, src_hbm)


NW = 32
ROWS_W = NSLOT // NW
DCHUNK = 32


@functools.lru_cache(maxsize=None)
def _dispatch_kernel_fn():
    return pl.kernel(
        _dispatch_body,
        out_type=jax.ShapeDtypeStruct((NSLOT, HIDDEN), jnp.float32),
        mesh=_sc_mesh(),
        scratch_types=[
            pltpu.VMEM((ROWS_W,), jnp.int32),
            pltpu.VMEM((DCHUNK, HIDDEN), jnp.float32),
            pltpu.VMEM((DCHUNK, HIDDEN), jnp.float32),
            pltpu.SemaphoreType.DMA,
            pltpu.SemaphoreType.DMA,
        ],
        compiler_params=pltpu.CompilerParams(needs_layout_passes=False),
    )


def _dispatch_body(xe_hbm, src_hbm, out_hbm, idx_v, rows_a, rows_b, sem_a, sem_b):
    cid = lax.axis_index("c")
    sid = lax.axis_index("s")
    wid = sid * 2 + cid
    base = wid * ROWS_W
    pltpu.sync_copy(src_hbm.at[pl.ds(base, ROWS_W)], idx_v)
    nrounds = ROWS_W // (2 * DCHUNK)
    for rr in range(nrounds):
        off = rr * 2 * DCHUNK
        cp_a = pltpu.async_copy(
            xe_hbm.at[idx_v.at[pl.ds(off, DCHUNK)]], rows_a, sem_a)
        cp_b = pltpu.async_copy(
            xe_hbm.at[idx_v.at[pl.ds(off + DCHUNK, DCHUNK)]], rows_b, sem_b)
        cp_a.wait()
        pltpu.sync_copy(rows_a, out_hbm.at[pl.ds(base + off, DCHUNK)])
        cp_b.wait()
        pltpu.sync_copy(rows_b, out_hbm.at[pl.ds(base + off + DCHUNK, DCHUNK)])


FBLK = 1024
NFB = FF // FBLK


def _ffn_body(d_ref, wg_ref, wu_ref, wd_ref, out_ref):
    j = pl.program_id(1)
    d = d_ref[0].astype(jnp.bfloat16)
    g = jnp.dot(d, wg_ref[0].astype(jnp.bfloat16), preferred_element_type=jnp.float32)
    u = jnp.dot(d, wu_ref[0].astype(jnp.bfloat16), preferred_element_type=jnp.float32)
    h = (jax.nn.gelu(g) * u).astype(jnp.bfloat16)
    o = jnp.dot(h, wd_ref[0].astype(jnp.bfloat16), preferred_element_type=jnp.float32)

    @pl.when(j == 0)
    def _():
        out_ref[0] = o

    @pl.when(j > 0)
    def _():
        out_ref[0] = out_ref[0] + o


def _ffn(disp, w_gate, w_up, w_down):
    return pl.pallas_call(
        _ffn_body,
        grid=(NUM_EXPERTS, NFB),
        in_specs=[
            pl.BlockSpec((1, CAP, HIDDEN), lambda e, j: (e, 0, 0)),
            pl.BlockSpec((1, HIDDEN, FBLK), lambda e, j: (e, 0, j)),
            pl.BlockSpec((1, HIDDEN, FBLK), lambda e, j: (e, 0, j)),
            pl.BlockSpec((1, FBLK, HIDDEN), lambda e, j: (e, j, 0)),
        ],
        out_specs=pl.BlockSpec((1, CAP, HIDDEN), lambda e, j: (e, 0, 0)),
        out_shape=jax.ShapeDtypeStruct((NUM_EXPERTS, CAP, HIDDEN), jnp.float32),
        compiler_params=pltpu.CompilerParams(
            dimension_semantics=("arbitrary", "arbitrary"),
        ),
    )(disp, w_gate, w_up, w_down)


TOK_W = T // NW
CCHUNK = 32


@functools.lru_cache(maxsize=None)
def _combine_kernel_fn():
    return pl.kernel(
        _combine_body,
        out_type=jax.ShapeDtypeStruct((T, HIDDEN), jnp.float32),
        mesh=_sc_mesh(),
        scratch_types=[
            pltpu.VMEM((TOK_W,), jnp.int32),
            pltpu.VMEM((TOK_W,), jnp.int32),
            pltpu.VMEM((TOK_W * 16,), jnp.float32),
            pltpu.VMEM((TOK_W * 16,), jnp.float32),
            pltpu.VMEM((CCHUNK, HIDDEN), jnp.float32),
            pltpu.VMEM((CCHUNK, HIDDEN), jnp.float32),
            pltpu.VMEM((CCHUNK, HIDDEN), jnp.float32),
            pltpu.SemaphoreType.DMA,
            pltpu.SemaphoreType.DMA,
        ],
        compiler_params=pltpu.CompilerParams(needs_layout_passes=False),
    )


def _combine_body(eo_hbm, slot0_hbm, slot1_hbm, w0_hbm, w1_hbm, y_hbm,
                  i0_v, i1_v, w0_v, w1_v, r0_v, r1_v, y_v, sem0, sem1):
    cid = lax.axis_index("c")
    sid = lax.axis_index("s")
    wid = sid * 2 + cid
    base = wid * TOK_W
    pltpu.sync_copy(slot0_hbm.at[pl.ds(base, TOK_W)], i0_v)
    pltpu.sync_copy(slot1_hbm.at[pl.ds(base, TOK_W)], i1_v)
    pltpu.sync_copy(w0_hbm.at[pl.ds(base * 16, TOK_W * 16)], w0_v)
    pltpu.sync_copy(w1_hbm.at[pl.ds(base * 16, TOK_W * 16)], w1_v)

    for cchunk in range(TOK_W // CCHUNK):
        c0 = cchunk * CCHUNK
        cp0 = pltpu.async_copy(eo_hbm.at[i0_v.at[pl.ds(c0, CCHUNK)]], r0_v, sem0)
        cp1 = pltpu.async_copy(eo_hbm.at[i1_v.at[pl.ds(c0, CCHUNK)]], r1_v, sem1)
        cp0.wait()
        cp1.wait()

        def tok(i, _):
            w0 = w0_v[pl.ds((c0 + i) * 16, 16)]
            w1 = w1_v[pl.ds((c0 + i) * 16, 16)]
            for jj in range(HIDDEN // 16):
                a = r0_v[i, pl.ds(jj * 16, 16)]
                b = r1_v[i, pl.ds(jj * 16, 16)]
                y_v[i, pl.ds(jj * 16, 16)] = w0 * a + w1 * b
            return 0

        lax.fori_loop(0, CCHUNK, tok, 0)
        pltpu.sync_copy(y_v, y_hbm.at[pl.ds(base + c0, CCHUNK)])


def kernel(x, gate_scale, Wg, pre_norm_gamma, W_gate, W_up, W_down):
    wg_pad = jnp.zeros((HIDDEN, LANES), jnp.float32).at[:, :NUM_EXPERTS].set(Wg)
    xe, meta = _router(x, gate_scale.reshape(1, HIDDEN), wg_pad,
                       pre_norm_gamma.reshape(1, HIDDEN))
    slot0 = meta[:, 0].astype(jnp.int32)
    slot1 = meta[:, 1].astype(jnp.int32)
    w0 = jnp.broadcast_to(meta[:, 2:3], (T, 16)).reshape(-1)
    w1 = jnp.broadcast_to(meta[:, 3:4], (T, 16)).reshape(-1)
    v0 = meta[:, 4].astype(jnp.int32)
    v1 = meta[:, 5].astype(jnp.int32)

    src = _slot_map_kernel_fn()(slot0, slot1, v0, v1)
    disp = _dispatch_kernel_fn()(xe, src)
    eo = _ffn(disp.reshape(NUM_EXPERTS, CAP, HIDDEN), W_gate, W_up, W_down)
    y = _combine_kernel_fn()(eo.reshape(NSLOT, HIDDEN), slot0, slot1, w0, w1)
    return y


# speedup vs baseline: 1.2619x; 1.0614x over previous
import functools

import jax
import jax.numpy as jnp
from jax import lax
from jax.experimental import pallas as pl
from jax.experimental.pallas import tpu as pltpu
from jax.experimental.pallas import tpu_sc as plsc

HIDDEN = 1024
NUM_EXPERTS = 8
TOPK = 2
FF = 2048
T = 2048
EPS = 1e-6
CAP = (T * TOPK) // NUM_EXPERTS
NSLOT = NUM_EXPERTS * CAP
XE_PAD = T + 16
LANES = 128
NMETA = 8


def _router_body(x_ref, gs_ref, wg_ref, gamma_ref, xe_ref, meta_ref):
    x = x_ref[...]
    ms = jnp.mean(x * x, axis=1, keepdims=True)
    r = lax.rsqrt(ms + EPS)
    xn = x * r
    xe_ref[pl.ds(0, T), :] = xn * gamma_ref[...]
    xe_ref[pl.ds(T, XE_PAD - T), :] = jnp.zeros((XE_PAD - T, HIDDEN), jnp.float32)

    xg = xn * (HIDDEN ** -0.5) * gs_ref[...]
    logits8 = jnp.dot(xg, wg_ref[...], preferred_element_type=jnp.float32)
    lg = jnp.concatenate(
        [logits8, jnp.full((T, LANES - NUM_EXPERTS), -1e30, jnp.float32)], axis=1)
    col = lax.broadcasted_iota(jnp.int32, (T, LANES), 1)
    m = jnp.max(lg, axis=1, keepdims=True)
    e = jnp.exp(lg - m)
    p = e / jnp.sum(e, axis=1, keepdims=True)

    m0 = jnp.max(p, axis=1, keepdims=True)
    i0 = jnp.min(jnp.where(p >= m0, col, LANES), axis=1, keepdims=True)
    p1 = jnp.where(col == i0, -1.0, p)
    m1 = jnp.max(p1, axis=1, keepdims=True)
    i1 = jnp.min(jnp.where(p1 >= m1, col, LANES), axis=1, keepdims=True)
    wsum = jnp.maximum(m0 + m1, 1e-20)
    w0 = m0 / wsum
    w1 = m1 / wsum

    cnt = (col == i0).astype(jnp.float32) + (col == i1).astype(jnp.float32)
    c = cnt
    sh = 1
    while sh < T:
        z = jnp.zeros((sh, LANES), jnp.float32)
        c = c + jnp.concatenate([z, c[: T - sh, :]], axis=0)
        sh *= 2
    excl = c - cnt

    pos0 = jnp.sum(jnp.where(col == i0, excl, 0.0), axis=1, keepdims=True)
    pos1 = jnp.sum(jnp.where(col == i1, excl, 0.0), axis=1, keepdims=True)
    keep0 = pos0 < CAP
    keep1 = pos1 < CAP
    f0 = i0.astype(jnp.float32)
    f1 = i1.astype(jnp.float32)
    s0_scat = jnp.where(keep0, f0 * CAP + pos0, float(NSLOT))
    s1_scat = jnp.where(keep1, f1 * CAP + pos1, float(NSLOT))
    s0_comb = f0 * CAP + jnp.minimum(pos0, CAP - 1)
    s1_comb = f1 * CAP + jnp.minimum(pos1, CAP - 1)
    w0 = jnp.where(keep0, w0, 0.0)
    w1 = jnp.where(keep1, w1, 0.0)

    col8 = lax.broadcasted_iota(jnp.int32, (T, NMETA), 1)
    meta = jnp.where(col8 == 0, s0_scat,
           jnp.where(col8 == 1, s1_scat,
           jnp.where(col8 == 2, s0_comb,
           jnp.where(col8 == 3, s1_comb,
           jnp.where(col8 == 4, w0,
           jnp.where(col8 == 5, w1, 0.0))))))
    meta_ref[...] = meta


def _router(x, gate_scale, wg, gamma):
    return pl.pallas_call(
        _router_body,
        out_shape=(
            jax.ShapeDtypeStruct((XE_PAD, HIDDEN), jnp.float32),
            jax.ShapeDtypeStruct((T, NMETA), jnp.float32),
        ),
    )(x, gate_scale, wg, gamma)


def _sc_mesh():
    return plsc.VectorSubcoreMesh(core_axis_name="c", subcore_axis_name="s")


NW = 32
ROWS_W = NSLOT // NW
DCHUNK = 32
DCH_N = ROWS_W // DCHUNK


@functools.lru_cache(maxsize=None)
def _dispatch_kernel_fn():
    return pl.kernel(
        _dispatch_body,
        out_type=jax.ShapeDtypeStruct((NSLOT, HIDDEN), jnp.float32),
        mesh=_sc_mesh(),
        scratch_types=[
            pltpu.VMEM((T,), jnp.int32),
            pltpu.VMEM((T,), jnp.int32),
            pltpu.VMEM((NSLOT,), jnp.int32),
            pltpu.VMEM((DCHUNK, HIDDEN), jnp.float32),
            pltpu.VMEM((DCHUNK, HIDDEN), jnp.float32),
            pltpu.VMEM((DCHUNK, HIDDEN), jnp.float32),
            pltpu.SemaphoreType.DMA,
            pltpu.SemaphoreType.DMA,
            pltpu.SemaphoreType.DMA,
            pltpu.SemaphoreType.DMA,
            pltpu.SemaphoreType.DMA,
            pltpu.SemaphoreType.DMA,
        ],
        compiler_params=pltpu.CompilerParams(needs_layout_passes=False),
    )


def _dispatch_body(xe_hbm, s0_hbm, s1_hbm, out_hbm,
                   s0_v, s1_v, src_v, r0, r1, r2, g0, g1, g2, w0, w1, w2):
    cid = lax.axis_index("c")
    sid = lax.axis_index("s")
    wid = sid * 2 + cid
    base = wid * ROWS_W
    pltpu.sync_copy(s0_hbm, s0_v)
    pltpu.sync_copy(s1_hbm, s1_v)

    for i in range(ROWS_W // 16):
        src_v[pl.ds(base + i * 16, 16)] = jnp.full((16,), T, jnp.int32)

    def scat(i, _):
        toks = lax.iota(jnp.int32, 16) + i * 16
        s0 = s0_v[pl.ds(i * 16, 16)]
        plsc.store_scatter(src_v, [s0], toks, mask=s0 < NSLOT)
        s1 = s1_v[pl.ds(i * 16, 16)]
        plsc.store_scatter(src_v, [s1], toks, mask=s1 < NSLOT)
        return 0

    lax.fori_loop(0, T // 16, scat, 0)

    bufs = (r0, r1, r2)
    gsems = (g0, g1, g2)
    wsems = (w0, w1, w2)

    def gather(ch):
        return pltpu.async_copy(
            xe_hbm.at[src_v.at[pl.ds(base + ch * DCHUNK, DCHUNK)]],
            bufs[ch % 3], gsems[ch % 3])

    cps = {ch: gather(ch) for ch in range(min(3, DCH_N))}
    wbs = {}
    for ch in range(DCH_N):
        cps[ch].wait()
        wbs[ch] = pltpu.async_copy(
            bufs[ch % 3], out_hbm.at[pl.ds(base + ch * DCHUNK, DCHUNK)],
            wsems[ch % 3])
        nxt = ch + 3
        if nxt < DCH_N:
            wbs[nxt - 3].wait()
            cps[nxt] = gather(nxt)
    for ch in range(max(0, DCH_N - 3), DCH_N):
        wbs[ch].wait()


FBLK = 1024
NFB = FF // FBLK


def _ffn_body(d_ref, wg_ref, wu_ref, wd_ref, out_ref):
    j = pl.program_id(1)
    d = d_ref[0].astype(jnp.bfloat16)
    g = jnp.dot(d, wg_ref[0].astype(jnp.bfloat16), preferred_element_type=jnp.float32)
    u = jnp.dot(d, wu_ref[0].astype(jnp.bfloat16), preferred_element_type=jnp.float32)
    h = (jax.nn.gelu(g) * u).astype(jnp.bfloat16)
    o = jnp.dot(h, wd_ref[0].astype(jnp.bfloat16), preferred_element_type=jnp.float32)

    @pl.when(j == 0)
    def _():
        out_ref[0] = o

    @pl.when(j > 0)
    def _():
        out_ref[0] = out_ref[0] + o


def _ffn(disp, w_gate, w_up, w_down):
    return pl.pallas_call(
        _ffn_body,
        grid=(NUM_EXPERTS, NFB),
        in_specs=[
            pl.BlockSpec((1, CAP, HIDDEN), lambda e, j: (e, 0, 0)),
            pl.BlockSpec((1, HIDDEN, FBLK), lambda e, j: (e, 0, j)),
            pl.BlockSpec((1, HIDDEN, FBLK), lambda e, j: (e, 0, j)),
            pl.BlockSpec((1, FBLK, HIDDEN), lambda e, j: (e, j, 0)),
        ],
        out_specs=pl.BlockSpec((1, CAP, HIDDEN), lambda e, j: (e, 0, 0)),
        out_shape=jax.ShapeDtypeStruct((NUM_EXPERTS, CAP, HIDDEN), jnp.float32),
        compiler_params=pltpu.CompilerParams(
            dimension_semantics=("arbitrary", "arbitrary"),
        ),
    )(disp, w_gate, w_up, w_down)


TOK_W = T // NW
CCHUNK = 16
CCH_N = TOK_W // CCHUNK


@functools.lru_cache(maxsize=None)
def _combine_kernel_fn():
    return pl.kernel(
        _combine_body,
        out_type=jax.ShapeDtypeStruct((T, HIDDEN), jnp.float32),
        mesh=_sc_mesh(),
        scratch_types=[
            pltpu.VMEM((TOK_W,), jnp.int32),
            pltpu.VMEM((TOK_W,), jnp.int32),
            pltpu.VMEM((TOK_W * 16,), jnp.float32),
            pltpu.VMEM((TOK_W * 16,), jnp.float32),
            pltpu.VMEM((CCHUNK, HIDDEN), jnp.float32),
            pltpu.VMEM((CCHUNK, HIDDEN), jnp.float32),
            pltpu.VMEM((CCHUNK, HIDDEN), jnp.float32),
            pltpu.VMEM((CCHUNK, HIDDEN), jnp.float32),
            pltpu.VMEM((CCHUNK, HIDDEN), jnp.float32),
            pltpu.VMEM((CCHUNK, HIDDEN), jnp.float32),
            pltpu.SemaphoreType.DMA,
            pltpu.SemaphoreType.DMA,
            pltpu.SemaphoreType.DMA,
            pltpu.SemaphoreType.DMA,
            pltpu.SemaphoreType.DMA,
            pltpu.SemaphoreType.DMA,
        ],
        compiler_params=pltpu.CompilerParams(needs_layout_passes=False),
    )


def _combine_body(eo_hbm, slot0_hbm, slot1_hbm, w0_hbm, w1_hbm, y_hbm,
                  i0_v, i1_v, w0_v, w1_v, r0a, r0b, r1a, r1b, ya, yb,
                  ga0, ga1, gb0, gb1, wa, wb):
    cid = lax.axis_index("c")
    sid = lax.axis_index("s")
    wid = sid * 2 + cid
    base = wid * TOK_W
    pltpu.sync_copy(slot0_hbm.at[pl.ds(base, TOK_W)], i0_v)
    pltpu.sync_copy(slot1_hbm.at[pl.ds(base, TOK_W)], i1_v)
    pltpu.sync_copy(w0_hbm.at[pl.ds(base * 16, TOK_W * 16)], w0_v)
    pltpu.sync_copy(w1_hbm.at[pl.ds(base * 16, TOK_W * 16)], w1_v)

    r0s = (r0a, r0b)
    r1s = (r1a, r1b)
    ys = (ya, yb)
    g0s = (ga0, gb0)
    g1s = (ga1, gb1)
    wsems = (wa, wb)

    def gather(ch):
        b = ch % 2
        c0 = ch * CCHUNK
        return (
            pltpu.async_copy(eo_hbm.at[i0_v.at[pl.ds(c0, CCHUNK)]], r0s[b], g0s[b]),
            pltpu.async_copy(eo_hbm.at[i1_v.at[pl.ds(c0, CCHUNK)]], r1s[b], g1s[b]),
        )

    cps = {ch: gather(ch) for ch in range(min(2, CCH_N))}
    wbs = {}
    for ch in range(CCH_N):
        b = ch % 2
        c0 = ch * CCHUNK
        cps[ch][0].wait()
        cps[ch][1].wait()
        if ch >= 2:
            wbs[ch - 2].wait()

        def tok(i, _):
            w0 = w0_v[pl.ds((c0 + i) * 16, 16)]
            w1 = w1_v[pl.ds((c0 + i) * 16, 16)]
            for jj in range(HIDDEN // 16):
                a = r0s[b][i, pl.ds(jj * 16, 16)]
                bb = r1s[b][i, pl.ds(jj * 16, 16)]
                ys[b][i, pl.ds(jj * 16, 16)] = w0 * a + w1 * bb
            return 0

        lax.fori_loop(0, CCHUNK, tok, 0)
        if ch + 2 < CCH_N:
            cps[ch + 2] = gather(ch + 2)
        wbs[ch] = pltpu.async_copy(
            ys[b], y_hbm.at[pl.ds(base + c0, CCHUNK)], wsems[b])
    for ch in range(max(0, CCH_N - 2), CCH_N):
        wbs[ch].wait()


def kernel(x, gate_scale, Wg, pre_norm_gamma, W_gate, W_up, W_down):
    xe, meta = _router(x, gate_scale.reshape(1, HIDDEN), Wg,
                       pre_norm_gamma.reshape(1, HIDDEN))
    s0_scat = meta[:, 0].astype(jnp.int32)
    s1_scat = meta[:, 1].astype(jnp.int32)
    s0_comb = meta[:, 2].astype(jnp.int32)
    s1_comb = meta[:, 3].astype(jnp.int32)
    w0 = jnp.broadcast_to(meta[:, 4:5], (T, 16)).reshape(-1)
    w1 = jnp.broadcast_to(meta[:, 5:6], (T, 16)).reshape(-1)

    disp = _dispatch_kernel_fn()(xe, s0_scat, s1_scat)
    eo = _ffn(disp.reshape(NUM_EXPERTS, CAP, HIDDEN), W_gate, W_up, W_down)
    y = _combine_kernel_fn()(eo.reshape(NSLOT, HIDDEN), s0_comb, s1_comb, w0, w1)
    return y
